# baseline (speedup 1.0000x reference)
import jax
import jax.numpy as jnp
from jax.experimental import pallas as pl

B, N, D = 4, 128, 256
K_TOP = 16
R = 128

_BF = jnp.bfloat16
_F32 = jnp.float32


_CHUNKS = 4
_RC = R // _CHUNKS


def _scores_kernel(s_full_ref, s_row_ref, w1_ref, b1_ref, w2_ref, sc_ref):
    sall = s_full_ref[0]
    w1 = w1_ref[...]
    b1 = b1_ref[...]
    w2 = w2_ref[...]

    scs = []
    for c in range(_CHUNKS):
        srow = s_row_ref[0, c * _RC:(c + 1) * _RC]
        si = jnp.broadcast_to(srow[:, None, :], (_RC, N, D))
        sj = jnp.broadcast_to(sall[None, :, :], (_RC, N, D))
        prod = si * sj
        pair = jnp.concatenate(
            [si.astype(_BF).reshape(_RC * N, D),
             sj.astype(_BF).reshape(_RC * N, D),
             prod.astype(_BF).reshape(_RC * N, D)], axis=-1)
        h = jax.lax.dot_general(pair, w1, (((1,), (0,)), ((), ())),
                                preferred_element_type=_F32)
        h = jnp.maximum(h + b1, 0.0).astype(_BF)
        sc = jax.lax.dot_general(h, w2, (((1,), (0,)), ((), ())),
                                 preferred_element_type=_F32)
        scs.append(sc.reshape(_RC, N))
    sc_ref[0] = jnp.concatenate(scs, axis=0)


def _topk_kernel(sc_ref, s_ref, ctx_ref, gate_ref, w_ref):
    scores = sc_ref[0]
    sall = s_ref[0]

    iota_l = jax.lax.broadcasted_iota(jnp.int32, (N, N), 1)
    cur = scores
    gate = jnp.zeros((N, N), _F32)
    for _ in range(K_TOP):
        m = jnp.max(cur, axis=1, keepdims=True)
        cand_idx = jnp.where(cur == m, iota_l, N)
        jmin = jnp.min(cand_idx, axis=1, keepdims=True)
        pick = cand_idx == jmin
        gate = gate + pick.astype(_F32)
        cur = jnp.where(pick, -jnp.inf, cur)

    wmat = gate * (1.0 / K_TOP)
    ctx = jax.lax.dot_general(wmat.astype(_BF), sall.astype(_BF),
                              (((1,), (0,)), ((), ())),
                              preferred_element_type=_F32)
    ctx_ref[0] = ctx
    gate_ref[0] = gate
    w_ref[0] = wmat


def kernel(s, W1, b1, W2, b2):
    del b2
    w1_bf = W1.astype(_BF)
    w2_bf = W2.astype(_BF)
    b1r = b1.reshape(1, 2 * D)

    nblk = N // R
    scores = pl.pallas_call(
        _scores_kernel,
        grid=(B, nblk),
        in_specs=[
            pl.BlockSpec((1, N, D), lambda b, r: (b, 0, 0)),
            pl.BlockSpec((1, R, D), lambda b, r: (b, r, 0)),
            pl.BlockSpec((3 * D, 2 * D), lambda b, r: (0, 0)),
            pl.BlockSpec((1, 2 * D), lambda b, r: (0, 0)),
            pl.BlockSpec((2 * D, 1), lambda b, r: (0, 0)),
        ],
        out_specs=pl.BlockSpec((1, R, N), lambda b, r: (b, r, 0)),
        out_shape=jax.ShapeDtypeStruct((B, N, N), jnp.float32),
    )(s, s, w1_bf, b1r, w2_bf)

    ctx, gate, w = pl.pallas_call(
        _topk_kernel,
        grid=(B,),
        in_specs=[
            pl.BlockSpec((1, N, N), lambda b: (b, 0, 0)),
            pl.BlockSpec((1, N, D), lambda b: (b, 0, 0)),
        ],
        out_specs=[
            pl.BlockSpec((1, N, D), lambda b: (b, 0, 0)),
            pl.BlockSpec((1, N, N), lambda b: (b, 0, 0)),
            pl.BlockSpec((1, N, N), lambda b: (b, 0, 0)),
        ],
        out_shape=(
            jax.ShapeDtypeStruct((B, N, D), jnp.float32),
            jax.ShapeDtypeStruct((B, N, N), jnp.float32),
            jax.ShapeDtypeStruct((B, N, N), jnp.float32),
        ),
    )(scores, s)
    return ctx, gate, w


# speedup vs baseline: 1.1298x; 1.1298x over previous
import jax
import jax.numpy as jnp
from jax.experimental import pallas as pl

B, N, D = 4, 128, 256
K_TOP = 16
R = 128

_BF = jnp.bfloat16
_F32 = jnp.float32


def _scores_kernel(s_full_ref, s_row_ref, w1_ref, b1_ref, w2_ref, sc_ref):
    srow = s_row_ref[0]
    sall = s_full_ref[0]
    w1 = w1_ref[...]
    b1 = b1_ref[...]
    w2 = w2_ref[...]

    si = jnp.broadcast_to(srow[:, None, :], (R, N, D))
    sj = jnp.broadcast_to(sall[None, :, :], (R, N, D))
    prod = si * sj
    pair = jnp.concatenate(
        [si.astype(_BF).reshape(R * N, D),
         sj.astype(_BF).reshape(R * N, D),
         prod.astype(_BF).reshape(R * N, D)], axis=-1)

    h = jax.lax.dot_general(pair, w1, (((1,), (0,)), ((), ())),
                            preferred_element_type=_F32)
    h = jnp.maximum(h + b1, 0.0).astype(_BF)
    sc = jax.lax.dot_general(h, w2, (((1,), (0,)), ((), ())),
                             preferred_element_type=_F32)
    sc_ref[0] = sc.reshape(R, N)


def _topk_kernel(sc_ref, s_ref, ctx_ref, gate_ref, w_ref):
    scores = sc_ref[...].reshape(B * N, N)
    iota_l = jax.lax.broadcasted_iota(jnp.int32, (B * N, N), 1)
    cur = scores
    gate = jnp.zeros((B * N, N), _F32)
    for _ in range(K_TOP):
        m = jnp.max(cur, axis=1, keepdims=True)
        cand_idx = jnp.where(cur == m, iota_l, N)
        jmin = jnp.min(cand_idx, axis=1, keepdims=True)
        pick = cand_idx == jmin
        gate = gate + pick.astype(_F32)
        cur = jnp.where(pick, -jnp.inf, cur)

    wmat = gate * (1.0 / K_TOP)
    wmat_bf = wmat.astype(_BF)
    for b in range(B):
        sall_bf = s_ref[b].astype(_BF)
        ctx_ref[b] = jax.lax.dot_general(
            wmat_bf[b * N:(b + 1) * N], sall_bf,
            (((1,), (0,)), ((), ())), preferred_element_type=_F32)
    gate_ref[...] = gate.reshape(B, N, N)
    w_ref[...] = wmat.reshape(B, N, N)


def kernel(s, W1, b1, W2, b2):
    del b2
    w1_bf = W1.astype(_BF)
    w2_bf = W2.astype(_BF)
    b1r = b1.reshape(1, 2 * D)

    nblk = N // R
    scores = pl.pallas_call(
        _scores_kernel,
        grid=(B, nblk),
        in_specs=[
            pl.BlockSpec((1, N, D), lambda b, r: (b, 0, 0)),
            pl.BlockSpec((1, R, D), lambda b, r: (b, r, 0)),
            pl.BlockSpec((3 * D, 2 * D), lambda b, r: (0, 0)),
            pl.BlockSpec((1, 2 * D), lambda b, r: (0, 0)),
            pl.BlockSpec((2 * D, 1), lambda b, r: (0, 0)),
        ],
        out_specs=pl.BlockSpec((1, R, N), lambda b, r: (b, r, 0)),
        out_shape=jax.ShapeDtypeStruct((B, N, N), jnp.float32),
    )(s, s, w1_bf, b1r, w2_bf)

    ctx, gate, w = pl.pallas_call(
        _topk_kernel,
        grid=(1,),
        in_specs=[
            pl.BlockSpec((B, N, N), lambda i: (0, 0, 0)),
            pl.BlockSpec((B, N, D), lambda i: (0, 0, 0)),
        ],
        out_specs=[
            pl.BlockSpec((B, N, D), lambda i: (0, 0, 0)),
            pl.BlockSpec((B, N, N), lambda i: (0, 0, 0)),
            pl.BlockSpec((B, N, N), lambda i: (0, 0, 0)),
        ],
        out_shape=(
            jax.ShapeDtypeStruct((B, N, D), jnp.float32),
            jax.ShapeDtypeStruct((B, N, N), jnp.float32),
            jax.ShapeDtypeStruct((B, N, N), jnp.float32),
        ),
    )(scores, s)
    return ctx, gate, w
